# baseline (speedup 1.0000x reference)
import functools

import jax
import jax.numpy as jnp
from jax.experimental import pallas as pl
from jax.experimental.pallas import tpu as pltpu

_B = 4096
_K = 200
_D = 64
_THR = 0.5
_EPS = 1e-12


def _row_block_kernel(q_ref, it_ref, bank_ref, out_bl_ref, out_sh_ref,
                      bsum_ref):
    pi = pl.program_id(0)

    q = q_ref[...]
    it = it_ref[...]
    bank = bank_ref[...]
    bank_t = jnp.transpose(bank, (0, 2, 1))

    it2 = jnp.sum(it * it, axis=-1, keepdims=True)
    q2 = jnp.sum(q * q, axis=-1, keepdims=True)
    qdoti = jnp.sum(q * it, axis=-1, keepdims=True)
    inv_q = 1.0 / jnp.clip(jnp.sqrt(q2), _EPS)
    inv_i = 1.0 / jnp.clip(jnp.sqrt(it2), _EPS)

    ns = jnp.sum(bank_t * bank_t, axis=1)
    dots_q = jnp.sum(bank_t * q[:, :, None], axis=1)
    dots_i = jnp.sum(bank_t * it[:, :, None], axis=1)

    inv_b = 1.0 / jnp.clip(jnp.sqrt(ns), _EPS)
    s_item = dots_i * inv_b * inv_i
    s_query = dots_q * inv_b * inv_q

    best = jnp.max(s_item, axis=-1, keepdims=True)
    kio = jax.lax.broadcasted_iota(jnp.int32, s_item.shape, 1)
    is_max = s_item == best
    j = jnp.min(jnp.where(is_max, kio, _K), axis=-1, keepdims=True)
    oh = kio == j
    ohf = oh.astype(jnp.float32)

    ns_j = jnp.sum(ns * ohf, axis=-1, keepdims=True)
    dq_j = jnp.sum(dots_q * ohf, axis=-1, keepdims=True)
    di_j = jnp.sum(dots_i * ohf, axis=-1, keepdims=True)

    m2 = 0.25 * (ns_j + 2.0 * di_j + it2)
    qdotm = 0.5 * (dq_j + qdoti)
    sim_j = qdotm * inv_q / jnp.clip(jnp.sqrt(m2), _EPS)

    logits_bl = jnp.where(oh, sim_j, s_query)
    mx = jnp.max(logits_bl, axis=-1, keepdims=True)
    e = jnp.exp(logits_bl - mx)
    w_bl = e / jnp.sum(e, axis=-1, keepdims=True)
    wj = jnp.sum(w_bl * ohf, axis=-1, keepdims=True)
    w_eff = w_bl - 0.5 * wj * ohf
    out_bl = (jnp.sum(w_eff[:, :, None] * bank, axis=1)
              + (0.5 * wj) * it)

    qi_cos = qdoti * inv_q * inv_i
    logits_sh = jnp.concatenate([s_query[:, 1:], qi_cos], axis=1)
    mxs = jnp.max(logits_sh, axis=-1, keepdims=True)
    es = jnp.exp(logits_sh - mxs)
    w_sh = es / jnp.sum(es, axis=-1, keepdims=True)
    w2 = jnp.concatenate(
        [jnp.zeros_like(w_sh[:, :1]), w_sh[:, :_K - 1]], axis=1)
    out_sh = (jnp.sum(w2[:, :, None] * bank, axis=1)
              + w_sh[:, _K - 1:] * it)

    out_bl_ref[...] = out_bl
    out_sh_ref[...] = out_sh

    @pl.when(pi == 0)
    def _init():
        bsum_ref[...] = jnp.zeros_like(bsum_ref)

    bsum_ref[...] += jnp.sum(best).reshape(1, 1)


@jax.jit
def kernel(query, item, memory_bank):
    bb = 32
    grid = (_B // bb,)
    out_bl, out_sh, bsum = pl.pallas_call(
        _row_block_kernel,
        grid=grid,
        in_specs=[
            pl.BlockSpec((bb, _D), lambda i: (i, 0)),
            pl.BlockSpec((bb, _D), lambda i: (i, 0)),
            pl.BlockSpec((bb, _K, _D), lambda i: (i, 0, 0)),
        ],
        out_specs=[
            pl.BlockSpec((bb, _D), lambda i: (i, 0)),
            pl.BlockSpec((bb, _D), lambda i: (i, 0)),
            pl.BlockSpec((1, 1), lambda i: (0, 0)),
        ],
        out_shape=[
            jax.ShapeDtypeStruct((_B, _D), jnp.float32),
            jax.ShapeDtypeStruct((_B, _D), jnp.float32),
            jax.ShapeDtypeStruct((1, 1), jnp.float32),
        ],
    )(query, item, memory_bank)
    mean_best = bsum[0, 0] / _B
    return jnp.where(mean_best >= _THR, out_bl, out_sh)


# speedup vs baseline: 1.1306x; 1.1306x over previous
import functools

import jax
import jax.numpy as jnp
from jax.experimental import pallas as pl
from jax.experimental.pallas import tpu as pltpu

_B = 4096
_K = 200
_D = 64
_THR = 0.5
_EPS = 1e-12


def _row_block_kernel(q_ref, it_ref, bank_ref, out_bl_ref, out_sh_ref,
                      bsum_ref):
    pi = pl.program_id(0)

    q = q_ref[...]
    it = it_ref[...]
    bank = bank_ref[...]
    bank_t = jnp.transpose(bank, (0, 2, 1))

    it2 = jnp.sum(it * it, axis=-1, keepdims=True)
    q2 = jnp.sum(q * q, axis=-1, keepdims=True)
    qdoti = jnp.sum(q * it, axis=-1, keepdims=True)
    inv_q = 1.0 / jnp.clip(jnp.sqrt(q2), _EPS)
    inv_i = 1.0 / jnp.clip(jnp.sqrt(it2), _EPS)

    ns = jnp.sum(bank_t * bank_t, axis=1)
    dots_q = jnp.sum(bank_t * q[:, :, None], axis=1)
    dots_i = jnp.sum(bank_t * it[:, :, None], axis=1)

    inv_b = 1.0 / jnp.clip(jnp.sqrt(ns), _EPS)
    s_item = dots_i * inv_b * inv_i
    s_query = dots_q * inv_b * inv_q

    best = jnp.max(s_item, axis=-1, keepdims=True)
    kio = jax.lax.broadcasted_iota(jnp.int32, s_item.shape, 1)
    is_max = s_item == best
    j = jnp.min(jnp.where(is_max, kio, _K), axis=-1, keepdims=True)
    oh = kio == j
    ohf = oh.astype(jnp.float32)

    ns_j = jnp.sum(ns * ohf, axis=-1, keepdims=True)
    dq_j = jnp.sum(dots_q * ohf, axis=-1, keepdims=True)
    di_j = jnp.sum(dots_i * ohf, axis=-1, keepdims=True)

    m2 = 0.25 * (ns_j + 2.0 * di_j + it2)
    qdotm = 0.5 * (dq_j + qdoti)
    sim_j = qdotm * inv_q / jnp.clip(jnp.sqrt(m2), _EPS)

    logits_bl = jnp.where(oh, sim_j, s_query)
    mx = jnp.max(logits_bl, axis=-1, keepdims=True)
    e = jnp.exp(logits_bl - mx)
    w_bl = e / jnp.sum(e, axis=-1, keepdims=True)
    wj = jnp.sum(w_bl * ohf, axis=-1, keepdims=True)
    w_eff = w_bl - 0.5 * wj * ohf
    out_bl = (jnp.sum(w_eff[:, :, None] * bank, axis=1)
              + (0.5 * wj) * it)

    qi_cos = qdoti * inv_q * inv_i
    logits_sh = jnp.concatenate([s_query[:, 1:], qi_cos], axis=1)
    mxs = jnp.max(logits_sh, axis=-1, keepdims=True)
    es = jnp.exp(logits_sh - mxs)
    w_sh = es / jnp.sum(es, axis=-1, keepdims=True)
    w2 = jnp.concatenate(
        [jnp.zeros_like(w_sh[:, :1]), w_sh[:, :_K - 1]], axis=1)
    out_sh = (jnp.sum(w2[:, :, None] * bank, axis=1)
              + w_sh[:, _K - 1:] * it)

    out_bl_ref[...] = out_bl
    out_sh_ref[...] = out_sh

    @pl.when(pi == 0)
    def _init():
        bsum_ref[...] = jnp.zeros_like(bsum_ref)

    bsum_ref[...] += jnp.sum(best).reshape(1, 1)


@jax.jit
def kernel(query, item, memory_bank):
    bb = 128
    grid = (_B // bb,)
    out_bl, out_sh, bsum = pl.pallas_call(
        _row_block_kernel,
        grid=grid,
        in_specs=[
            pl.BlockSpec((bb, _D), lambda i: (i, 0)),
            pl.BlockSpec((bb, _D), lambda i: (i, 0)),
            pl.BlockSpec((bb, _K, _D), lambda i: (i, 0, 0)),
        ],
        out_specs=[
            pl.BlockSpec((bb, _D), lambda i: (i, 0)),
            pl.BlockSpec((bb, _D), lambda i: (i, 0)),
            pl.BlockSpec((1, 1), lambda i: (0, 0)),
        ],
        out_shape=[
            jax.ShapeDtypeStruct((_B, _D), jnp.float32),
            jax.ShapeDtypeStruct((_B, _D), jnp.float32),
            jax.ShapeDtypeStruct((1, 1), jnp.float32),
        ],
    )(query, item, memory_bank)
    mean_best = bsum[0, 0] / _B
    return jnp.where(mean_best >= _THR, out_bl, out_sh)


# speedup vs baseline: 1.2989x; 1.1489x over previous
import jax
import jax.numpy as jnp
from jax.experimental import pallas as pl
from jax.experimental.pallas import tpu as pltpu

_B = 4096
_K = 200
_D = 64
_THR = 0.5
_EPS = 1e-12
_BB = 128
_NB = _B // _BB


def _two_phase_kernel(q_ref, it_ref, bank_ref, out_ref,
                      weff_ref, w2_ref, coef_ref, bsum_ref):
    s = pl.program_id(0)

    @pl.when(s < _NB)
    def _phase_a():
        q = q_ref[...]
        it = it_ref[...]
        bank = bank_ref[...]
        bank_t = jnp.transpose(bank, (0, 2, 1))

        it2 = jnp.sum(it * it, axis=-1, keepdims=True)
        q2 = jnp.sum(q * q, axis=-1, keepdims=True)
        qdoti = jnp.sum(q * it, axis=-1, keepdims=True)
        inv_q = 1.0 / jnp.clip(jnp.sqrt(q2), _EPS)
        inv_i = 1.0 / jnp.clip(jnp.sqrt(it2), _EPS)

        ns = jnp.sum(bank_t * bank_t, axis=1)
        dots_q = jnp.sum(bank_t * q[:, :, None], axis=1)
        dots_i = jnp.sum(bank_t * it[:, :, None], axis=1)

        inv_b = 1.0 / jnp.clip(jnp.sqrt(ns), _EPS)
        s_item = dots_i * inv_b * inv_i
        s_query = dots_q * inv_b * inv_q

        best = jnp.max(s_item, axis=-1, keepdims=True)
        kio = jax.lax.broadcasted_iota(jnp.int32, s_item.shape, 1)
        is_max = s_item == best
        j = jnp.min(jnp.where(is_max, kio, _K), axis=-1, keepdims=True)
        oh = kio == j
        ohf = oh.astype(jnp.float32)

        ns_j = jnp.sum(ns * ohf, axis=-1, keepdims=True)
        dq_j = jnp.sum(dots_q * ohf, axis=-1, keepdims=True)
        di_j = jnp.sum(dots_i * ohf, axis=-1, keepdims=True)

        m2 = 0.25 * (ns_j + 2.0 * di_j + it2)
        qdotm = 0.5 * (dq_j + qdoti)
        sim_j = qdotm * inv_q / jnp.clip(jnp.sqrt(m2), _EPS)

        logits_bl = jnp.where(oh, sim_j, s_query)
        mx = jnp.max(logits_bl, axis=-1, keepdims=True)
        e = jnp.exp(logits_bl - mx)
        w_bl = e / jnp.sum(e, axis=-1, keepdims=True)
        wj = jnp.sum(w_bl * ohf, axis=-1, keepdims=True)
        w_eff = w_bl - 0.5 * wj * ohf

        qi_cos = qdoti * inv_q * inv_i
        logits_sh = jnp.concatenate([s_query[:, 1:], qi_cos], axis=1)
        mxs = jnp.max(logits_sh, axis=-1, keepdims=True)
        es = jnp.exp(logits_sh - mxs)
        w_sh = es / jnp.sum(es, axis=-1, keepdims=True)
        w2 = jnp.concatenate(
            [jnp.zeros_like(w_sh[:, :1]), w_sh[:, :_K - 1]], axis=1)

        row0 = s * _BB
        weff_ref[pl.ds(row0, _BB), :] = w_eff
        w2_ref[pl.ds(row0, _BB), :] = w2
        coef_ref[pl.ds(row0, _BB), 0:1] = 0.5 * wj
        coef_ref[pl.ds(row0, _BB), 1:2] = w_sh[:, _K - 1:]

        @pl.when(s == 0)
        def _init():
            bsum_ref[...] = jnp.zeros_like(bsum_ref)

        bsum_ref[...] += jnp.sum(best).reshape(1, 1)

    @pl.when(s >= _NB)
    def _phase_b():
        row0 = (s - _NB) * _BB
        sel = bsum_ref[0, 0] >= _THR * _B
        w = jnp.where(sel, weff_ref[pl.ds(row0, _BB), :],
                      w2_ref[pl.ds(row0, _BB), :])
        c = jnp.where(sel, coef_ref[pl.ds(row0, _BB), 0:1],
                      coef_ref[pl.ds(row0, _BB), 1:2])
        bank = bank_ref[...]
        it = it_ref[...]
        out_ref[...] = jnp.sum(w[:, :, None] * bank, axis=1) + c * it


@jax.jit
def kernel(query, item, memory_bank):
    grid = (2 * _NB,)
    blk = lambda s: (jnp.where(s < _NB, s, s - _NB), 0)
    blk3 = lambda s: (jnp.where(s < _NB, s, s - _NB), 0, 0)
    out = pl.pallas_call(
        _two_phase_kernel,
        grid=grid,
        in_specs=[
            pl.BlockSpec((_BB, _D), blk),
            pl.BlockSpec((_BB, _D), blk),
            pl.BlockSpec((_BB, _K, _D), blk3),
        ],
        out_specs=pl.BlockSpec((_BB, _D), lambda s: (jnp.maximum(s - _NB, 0), 0)),
        out_shape=jax.ShapeDtypeStruct((_B, _D), jnp.float32),
        scratch_shapes=[
            pltpu.VMEM((_B, _K), jnp.float32),
            pltpu.VMEM((_B, _K), jnp.float32),
            pltpu.VMEM((_B, 2), jnp.float32),
            pltpu.VMEM((1, 1), jnp.float32),
        ],
    )(query, item, memory_bank)
    return out
